# Initial kernel scaffold; baseline (speedup 1.0000x reference)
import jax
import jax.numpy as jnp
from jax.experimental import pallas as pl


def kernel(x, edge_index, W1, b1, W2, b2, W3, b3):
    raise NotImplementedError("write your pallas kernel here")


# speedup vs baseline: 8.4129x; 8.4129x over previous
import functools

import jax
import jax.numpy as jnp
from jax import lax
from jax.experimental import pallas as pl
from jax.experimental.pallas import tpu as pltpu
from jax.experimental.pallas import tpu_sc as plsc

NC = 2
NS = 16
NW = NC * NS
EB = 128


def _ceil_to(a, m):
    return (a + m - 1) // m * m


def _deg_body(steps, dst_hbm, zeros_hbm, out_hbm, deg_sh, didx_v, ones_v, sem):
    npad = deg_sh.shape[0]
    c = lax.axis_index("c")
    s = lax.axis_index("s")
    rows = npad // NS
    base_r = s * rows
    pltpu.sync_copy(zeros_hbm.at[pl.ds(base_r, rows)], deg_sh.at[pl.ds(base_r, rows)])
    def fill(k, carry):
        ones_v[pl.ds(k * 16, 16)] = jnp.ones((16,), jnp.float32)
        return carry
    lax.fori_loop(0, EB // 16, fill, 0)
    plsc.subcore_barrier()

    w = s * NC + c
    base_e = w * steps * EB

    def body(j, carry):
        pltpu.sync_copy(dst_hbm.at[pl.ds(base_e + j * EB, EB)], didx_v)
        pltpu.sync_copy(ones_v, deg_sh.at[didx_v], add=True)
        return carry
    lax.fori_loop(0, steps, body, 0)
    plsc.subcore_barrier()
    pltpu.sync_copy(deg_sh.at[pl.ds(base_r, rows)], out_hbm.at[c, pl.ds(base_r, rows)])


def _agg_body(steps, g_hbm, src_hbm, dst_hbm, zeros_hbm, out_hbm,
              acc_sh, sidx_v, didx_v, rows_v, sem):
    npad, d = acc_sh.shape
    c = lax.axis_index("c")
    s = lax.axis_index("s")
    rows = npad // NS
    base_r = s * rows
    @pl.when(c == 0)
    def _():
        pltpu.sync_copy(g_hbm.at[pl.ds(base_r, rows)], acc_sh.at[pl.ds(base_r, rows)])
    @pl.when(c != 0)
    def _():
        pltpu.sync_copy(zeros_hbm.at[pl.ds(base_r, rows)], acc_sh.at[pl.ds(base_r, rows)])
    plsc.subcore_barrier()

    w = s * NC + c
    base_e = w * steps * EB

    def body(j, carry):
        off = base_e + j * EB
        pltpu.sync_copy(src_hbm.at[pl.ds(off, EB)], sidx_v)
        pltpu.sync_copy(dst_hbm.at[pl.ds(off, EB)], didx_v)
        pltpu.async_copy(g_hbm.at[sidx_v], rows_v, sem).wait()
        pltpu.sync_copy(rows_v, acc_sh.at[didx_v], add=True)
        return carry
    lax.fori_loop(0, steps, body, 0)
    plsc.subcore_barrier()
    pltpu.sync_copy(acc_sh.at[pl.ds(base_r, rows)],
                    out_hbm.at[c, pl.ds(base_r, rows)])


def _dinv_body(degs_ref, out_ref):
    d = degs_ref[0, :] + degs_ref[1, :] + 1.0
    out_ref[...] = lax.rsqrt(d)


def _mm_first_body(x_ref, w_ref, dinv_ref, g_ref):
    h = jnp.dot(x_ref[...], w_ref[...], preferred_element_type=jnp.float32)
    g_ref[...] = h * dinv_ref[...]


def _mm_mid_body(p0_ref, p1_ref, b_ref, dinv_ref, w_ref, g_ref):
    z = jnp.maximum((p0_ref[...] + p1_ref[...]) * dinv_ref[...] + b_ref[...], 0.0)
    h = jnp.dot(z, w_ref[...], preferred_element_type=jnp.float32)
    g_ref[...] = h * dinv_ref[...]


def _mm_last_body(p0_ref, p1_ref, b_ref, dinv_ref, out_ref):
    out_ref[...] = (p0_ref[...] + p1_ref[...]) * dinv_ref[...] + b_ref[...]


def kernel(x, edge_index, W1, b1, W2, b2, W3, b3):
    n, d = x.shape
    e = edge_index.shape[1]
    npad = _ceil_to(n + 1, 2048)
    epad = _ceil_to(e, NW * EB)
    steps = epad // (NW * EB)
    R = 2048

    src = jnp.concatenate([edge_index[0], jnp.zeros((epad - e,), jnp.int32)])
    dst = jnp.concatenate([edge_index[1], jnp.full((epad - e,), n, jnp.int32)])
    xp = jnp.pad(x, ((0, npad - n), (0, 0)))
    zeros2d = jnp.zeros((npad, d), jnp.float32)
    zeros1d = jnp.zeros((npad,), jnp.float32)

    mesh = plsc.VectorSubcoreMesh(core_axis_name="c", subcore_axis_name="s",
                                  num_cores=NC, num_subcores=NS)

    deg_call = pl.kernel(
        functools.partial(_deg_body, steps),
        out_type=jax.ShapeDtypeStruct((NC, npad), jnp.float32),
        mesh=mesh,
        scratch_types=[
            pltpu.VMEM_SHARED((npad,), jnp.float32),
            pltpu.VMEM((EB,), jnp.int32),
            pltpu.VMEM((EB,), jnp.float32),
            pltpu.SemaphoreType.DMA,
        ],
    )

    agg_call = pl.kernel(
        functools.partial(_agg_body, steps),
        out_type=jax.ShapeDtypeStruct((NC, npad, d), jnp.float32),
        mesh=mesh,
        scratch_types=[
            pltpu.VMEM_SHARED((npad, d), jnp.float32),
            pltpu.VMEM((EB,), jnp.int32),
            pltpu.VMEM((EB,), jnp.int32),
            pltpu.VMEM((EB, d), jnp.float32),
            pltpu.SemaphoreType.DMA,
        ],
    )

    dinv_call = pl.pallas_call(
        _dinv_body,
        out_shape=jax.ShapeDtypeStruct((npad,), jnp.float32),
    )

    row_spec = pl.BlockSpec((R, d), lambda i: (i, 0))
    w_spec = pl.BlockSpec((d, d), lambda i: (0, 0))
    b_spec = pl.BlockSpec((1, d), lambda i: (0, 0))
    dinv_spec = pl.BlockSpec((R, 1), lambda i: (i, 0))
    grid = (npad // R,)
    gshape = jax.ShapeDtypeStruct((npad, d), jnp.float32)

    mm_first = pl.pallas_call(
        _mm_first_body, grid=grid,
        in_specs=[row_spec, w_spec, dinv_spec],
        out_specs=row_spec, out_shape=gshape,
    )
    mm_mid = pl.pallas_call(
        _mm_mid_body, grid=grid,
        in_specs=[row_spec, row_spec, b_spec, dinv_spec, w_spec],
        out_specs=row_spec, out_shape=gshape,
    )
    mm_last = pl.pallas_call(
        _mm_last_body, grid=grid,
        in_specs=[row_spec, row_spec, b_spec, dinv_spec],
        out_specs=row_spec, out_shape=gshape,
    )

    degs = deg_call(dst, zeros1d)
    dinv_col = dinv_call(degs).reshape(npad, 1)
    b1r = b1.reshape(1, d)
    b2r = b2.reshape(1, d)
    b3r = b3.reshape(1, d)

    g = mm_first(xp, W1, dinv_col)
    p = agg_call(g, src, dst, zeros2d)
    g = mm_mid(p[0], p[1], b1r, dinv_col, W2)
    p = agg_call(g, src, dst, zeros2d)
    g = mm_mid(p[0], p[1], b2r, dinv_col, W3)
    p = agg_call(g, src, dst, zeros2d)
    out = mm_last(p[0], p[1], b3r, dinv_col)
    return out[:n]
